# baseline (speedup 1.0000x reference)
import functools
import math

import jax
import jax.numpy as jnp
from jax import lax
from jax.experimental import pallas as pl
from jax.experimental.pallas import tpu as pltpu
from jax.experimental.pallas import tpu_sc as plsc

N_NODES = 10000
N_EDGES = 160000
D = 256
DH = 128
NC = 2
NS = 16
NW = NC * NS
L = 16

C1 = 64
K1 = 80
NB1 = 3
EW1 = K1 * C1
E_PAD = NW * EW1
C2 = 128
N_CH2 = N_EDGES // C2
RCP = 624

_MESH = plsc.VectorSubcoreMesh(core_axis_name="c", subcore_axis_name="s")

_GDN = lax.GatherDimensionNumbers(
    offset_dims=(), collapsed_slice_dims=(0,), start_index_map=(0,))


def _vgather(x, idx):
    return lax.gather(x, idx[:, None], _GDN, (1,),
                      mode=lax.GatherScatterMode.PROMISE_IN_BOUNDS)


def _dense_in_body(x_ref, wqk_ref, bqk_ref, wv_ref, bv_ref,
                   qkb_ref, vlo_ref, vhi_ref):
    x = x_ref[...]
    qk = jnp.dot(x, wqk_ref[...],
                 preferred_element_type=jnp.float32) + bqk_ref[...]
    qkb_ref[...] = qk.astype(jnp.bfloat16)
    v = jnp.dot(x, wv_ref[...], preferred_element_type=jnp.float32) + bv_ref[...]
    vlo_ref[...] = v[:, :DH]
    vhi_ref[...] = v[:, DH:]


def _dense_in(x, Wqk, bqk, Wv, bv):
    blk = 400
    grid = (N_NODES // blk,)
    return pl.pallas_call(
        _dense_in_body,
        grid=grid,
        in_specs=[
            pl.BlockSpec((blk, D), lambda i: (i, 0)),
            pl.BlockSpec((D, D), lambda i: (0, 0)),
            pl.BlockSpec((1, D), lambda i: (0, 0)),
            pl.BlockSpec((D, D), lambda i: (0, 0)),
            pl.BlockSpec((1, D), lambda i: (0, 0)),
        ],
        out_specs=[
            pl.BlockSpec((blk, D), lambda i: (i, 0)),
            pl.BlockSpec((blk, DH), lambda i: (i, 0)),
            pl.BlockSpec((blk, DH), lambda i: (i, 0)),
        ],
        out_shape=[
            jax.ShapeDtypeStruct((N_NODES, D), jnp.bfloat16),
            jax.ShapeDtypeStruct((N_NODES, DH), jnp.float32),
            jax.ShapeDtypeStruct((N_NODES, DH), jnp.float32),
        ],
    )(x, Wqk, bqk.reshape(1, D), Wv, bv.reshape(1, D))


def _scores_body(qkb_hbm, dst_hbm, src_hbm, ex_hbm,
                 idxd, idxs, qbuf, kbuf, exall, semq, semk):
    wid = lax.axis_index("s") * NC + lax.axis_index("c")
    inv_scale = 1.0 / math.sqrt(D)
    lane = lax.iota(jnp.int32, L)

    pltpu.sync_copy(dst_hbm.at[pl.ds(wid * EW1, EW1)], idxd)
    pltpu.sync_copy(src_hbm.at[pl.ds(wid * EW1, EW1)], idxs)

    def fetch(k, b):
        off = pl.multiple_of(b * C1, 8)
        ioff = k * C1
        pltpu.async_copy(qkb_hbm.at[idxd.at[pl.ds(ioff, C1)]],
                         qbuf.at[pl.ds(off, C1)], semq.at[b])
        pltpu.async_copy(qkb_hbm.at[idxs.at[pl.ds(ioff, C1)]],
                         kbuf.at[pl.ds(off, C1)], semk.at[b])

    fetch(0, 0)
    fetch(1, 1)

    def chunk_step(k, _):
        b = lax.rem(k, NB1)

        @pl.when(k + 2 < K1)
        def _prefetch():
            fetch(k + 2, lax.rem(k + 2, NB1))

        off = pl.multiple_of(b * C1, 8)
        ioff = k * C1
        pltpu.make_async_copy(qkb_hbm.at[idxd.at[pl.ds(ioff, C1)]],
                              qbuf.at[pl.ds(off, C1)], semq.at[b]).wait()
        pltpu.make_async_copy(qkb_hbm.at[idxs.at[pl.ds(ioff, C1)]],
                              kbuf.at[pl.ds(off, C1)], semk.at[b]).wait()

        def grp_step(g, _):
            sv = jnp.zeros((L,), jnp.float32)
            for i in range(L):
                e = off + g * L + i
                terms = []
                for j in range(DH // L):
                    qi = qbuf[e, pl.ds(j * L, L)]
                    ki = kbuf[e, pl.ds(j * L, L)]
                    qa = lax.bitcast_convert_type(qi << 16, jnp.float32)
                    qb = lax.bitcast_convert_type(qi & jnp.int32(-65536),
                                                  jnp.float32)
                    ka = lax.bitcast_convert_type(ki << 16, jnp.float32)
                    kb = lax.bitcast_convert_type(ki & jnp.int32(-65536),
                                                  jnp.float32)
                    terms += [qa * ka, qb * kb]
                while len(terms) > 1:
                    terms = [terms[t] + terms[t + 1]
                             for t in range(0, len(terms), 2)]
                acc = terms[0]
                for sh in (8, 4, 2, 1):
                    acc = acc + _vgather(acc, lane ^ sh)
                sv = jnp.where(lane == i, acc * inv_scale, sv)
            sv = jnp.minimum(jnp.maximum(sv, -5.0), 5.0)
            exall[pl.ds(k * C1 + g * L, L)] = jnp.exp(sv)
            return 0

        lax.fori_loop(0, C1 // L, grp_step, 0)
        return 0

    lax.fori_loop(0, K1, chunk_step, 0)
    pltpu.sync_copy(exall, ex_hbm.at[pl.ds(wid * EW1, EW1)])


def _edge_scores(qkb, dstp, srcp):
    kern = pl.kernel(
        _scores_body,
        out_type=jax.ShapeDtypeStruct((E_PAD,), jnp.float32),
        mesh=_MESH,
        scratch_types=[
            pltpu.VMEM((EW1,), jnp.int32),
            pltpu.VMEM((EW1,), jnp.int32),
            pltpu.VMEM((NB1 * C1, D // 2), jnp.int32),
            pltpu.VMEM((NB1 * C1, D // 2), jnp.int32),
            pltpu.VMEM((EW1,), jnp.float32),
            pltpu.SemaphoreType.DMA((NB1,)),
            pltpu.SemaphoreType.DMA((NB1,)),
        ],
    )
    return kern(qkb, dstp, srcp)


def _aggregate_body(vlo_hbm, vhi_hbm, dst_hbm, src_hbm, ex_hbm, zrow_hbm,
                    zseg_hbm, raw_hbm, seg_hbm,
                    idxd, idxs, exb, vbuf, segbuf, raw_sp, seg_sp,
                    semidx, semv, semsc, semseg):
    c = lax.axis_index("c")
    sid = lax.axis_index("s")
    lane = lax.iota(jnp.int32, L)

    pltpu.sync_copy(zrow_hbm, raw_sp.at[pl.ds(sid * RCP, RCP)])

    @pl.when(sid == 0)
    def _zero_tail():
        pltpu.sync_copy(zrow_hbm.at[pl.ds(0, N_NODES - NS * RCP)],
                        raw_sp.at[pl.ds(NS * RCP, N_NODES - NS * RCP)])

    pltpu.sync_copy(zseg_hbm, segbuf.at[0])

    @pl.when(sid < 10)
    def _zero_seg():
        pltpu.sync_copy(segbuf.at[0], seg_sp.at[pl.ds(sid * 1024, 1024)])
    plsc.subcore_barrier()

    n_ch = jnp.where(sid < N_CH2 % NS, N_CH2 // NS + 1, N_CH2 // NS)

    def idx_copies(k, s):
        base = (sid + k * NS) * C2
        return (
            (dst_hbm.at[pl.ds(base, C2)], idxd.at[s]),
            (src_hbm.at[pl.ds(base, C2)], idxs.at[s]),
            (ex_hbm.at[pl.ds(base, C2)], exb.at[s]),
        )

    def fetch_idx(k):
        s = lax.rem(k, 4)
        for a, bref in idx_copies(k, s):
            pltpu.async_copy(a, bref, semidx.at[s])

    def wait_idx(k):
        s = lax.rem(k, 4)
        for a, bref in idx_copies(k, s):
            pltpu.make_async_copy(a, bref, semidx.at[s]).wait()

    def fetch_v(k, b):
        s = lax.rem(k, 4)
        off = pl.multiple_of(b * C2, 8)

        @pl.when(c == 0)
        def _gather_lo():
            pltpu.async_copy(vlo_hbm.at[idxs.at[s]],
                             vbuf.at[pl.ds(off, C2)], semv.at[b])

        @pl.when(c == 1)
        def _gather_hi():
            pltpu.async_copy(vhi_hbm.at[idxs.at[s]],
                             vbuf.at[pl.ds(off, C2)], semv.at[b])

    fetch_idx(0)
    fetch_idx(1)
    wait_idx(0)
    fetch_v(0, 0)

    def chunk_step(k, _):
        b = lax.rem(k, 2)
        nb = 1 - b
        off = pl.multiple_of(b * C2, 8)
        s = lax.rem(k, 4)

        @pl.when(k >= 1)
        def _drain_prev():
            noff = pl.multiple_of(nb * C2, 8)
            sp = lax.rem(k - 1, 4)
            pltpu.make_async_copy(vbuf.at[pl.ds(noff, C2)],
                                  raw_sp.at[idxd.at[sp]], semsc.at[nb]).wait()

            @pl.when(c == 0)
            def _drain_seg():
                pltpu.make_async_copy(exb.at[sp], seg_sp.at[idxd.at[sp]],
                                      semseg.at[nb]).wait()

        @pl.when(k + 2 < n_ch)
        def _pf_idx():
            fetch_idx(k + 2)

        @pl.when(k + 1 < n_ch)
        def _pf_v():
            wait_idx(k + 1)
            fetch_v(k + 1, nb)

        pltpu.make_async_copy(vlo_hbm.at[idxs.at[s]],
                              vbuf.at[pl.ds(off, C2)], semv.at[b]).wait()

        def scale_step(g, _):
            exv = exb[s, pl.ds(g * L, L)]
            for i in range(L):
                e = off + g * L + i
                sc = _vgather(exv, jnp.full((L,), i, jnp.int32))
                for j in range(DH // L):
                    vbuf[e, pl.ds(j * L, L)] = vbuf[e, pl.ds(j * L, L)] * sc
            return 0

        lax.fori_loop(0, C2 // L, scale_step, 0)

        pltpu.async_copy(vbuf.at[pl.ds(off, C2)], raw_sp.at[idxd.at[s]],
                         semsc.at[b], add=True)

        @pl.when(c == 0)
        def _seg_add():
            pltpu.async_copy(exb.at[s], seg_sp.at[idxd.at[s]],
                             semseg.at[b], add=True)
        return 0

    lax.fori_loop(0, n_ch, chunk_step, 0)

    bl = lax.rem(n_ch - 1, 2)
    sl = lax.rem(n_ch - 1, 4)
    loff = pl.multiple_of(bl * C2, 8)
    pltpu.make_async_copy(vbuf.at[pl.ds(loff, C2)],
                          raw_sp.at[idxd.at[sl]], semsc.at[bl]).wait()

    @pl.when(c == 0)
    def _drain_seg_last():
        pltpu.make_async_copy(exb.at[sl], seg_sp.at[idxd.at[sl]],
                              semseg.at[bl]).wait()

    plsc.subcore_barrier()

    pltpu.sync_copy(raw_sp.at[pl.ds(sid * RCP, RCP)],
                    raw_hbm.at[pl.ds(c * N_NODES + sid * RCP, RCP)])

    @pl.when(sid == 0)
    def _raw_tail():
        pltpu.sync_copy(raw_sp.at[pl.ds(NS * RCP, N_NODES - NS * RCP)],
                        raw_hbm.at[pl.ds(c * N_NODES + NS * RCP,
                                         N_NODES - NS * RCP)])

    @pl.when((c == 0) & (sid < 10))
    def _seg_out():
        pltpu.sync_copy(seg_sp.at[pl.ds(sid * 1024, 1024)], segbuf.at[0])
        pltpu.sync_copy(segbuf.at[0], seg_hbm.at[pl.ds(sid * 1024, 1024)])


def _aggregate(vlo, vhi, dst1, src1, ex1, zrow, zseg):
    kern = pl.kernel(
        _aggregate_body,
        out_type=[
            jax.ShapeDtypeStruct((2 * N_NODES, DH), jnp.float32),
            jax.ShapeDtypeStruct((10240,), jnp.float32),
        ],
        mesh=_MESH,
        scratch_types=[
            pltpu.VMEM((4, C2), jnp.int32),
            pltpu.VMEM((4, C2), jnp.int32),
            pltpu.VMEM((4, C2), jnp.float32),
            pltpu.VMEM((2 * C2, DH), jnp.float32),
            pltpu.VMEM((1, 1024), jnp.float32),
            pltpu.VMEM_SHARED((N_NODES, DH), jnp.float32),
            pltpu.VMEM_SHARED((10240,), jnp.float32),
            pltpu.SemaphoreType.DMA((4,)),
            pltpu.SemaphoreType.DMA((2,)),
            pltpu.SemaphoreType.DMA((2,)),
            pltpu.SemaphoreType.DMA((2,)),
        ],
    )
    return kern(vlo, vhi, dst1, src1, ex1, zrow, zseg)


def _dense_out_body(rawlo_ref, rawhi_ref, seg_ref, x_ref, wo_ref, bo_ref,
                    gamma_ref, beta_ref, out_ref):
    seg = jnp.maximum(seg_ref[...], 1e-30)
    agg = jnp.concatenate([rawlo_ref[...], rawhi_ref[...]], axis=1) / seg
    h = jnp.dot(agg, wo_ref[...], preferred_element_type=jnp.float32)
    h = h + bo_ref[...] + x_ref[...]
    mu = jnp.mean(h, axis=-1, keepdims=True)
    d = h - mu
    var = jnp.mean(d * d, axis=-1, keepdims=True)
    out_ref[...] = d * jax.lax.rsqrt(var + 1e-5) * gamma_ref[...] + beta_ref[...]


def _dense_out(rawlo, rawhi, seg, x, Wo, bo, gamma, beta):
    blk = 400
    grid = (N_NODES // blk,)
    return pl.pallas_call(
        _dense_out_body,
        grid=grid,
        in_specs=[
            pl.BlockSpec((blk, DH), lambda i: (i, 0)),
            pl.BlockSpec((blk, DH), lambda i: (i, 0)),
            pl.BlockSpec((blk, 1), lambda i: (i, 0)),
            pl.BlockSpec((blk, D), lambda i: (i, 0)),
            pl.BlockSpec((D, D), lambda i: (0, 0)),
            pl.BlockSpec((1, D), lambda i: (0, 0)),
            pl.BlockSpec((1, D), lambda i: (0, 0)),
            pl.BlockSpec((1, D), lambda i: (0, 0)),
        ],
        out_specs=pl.BlockSpec((blk, D), lambda i: (i, 0)),
        out_shape=jax.ShapeDtypeStruct((N_NODES, D), jnp.float32),
    )(rawlo, rawhi, seg.reshape(N_NODES, 1), x, Wo, bo.reshape(1, D),
      gamma.reshape(1, D), beta.reshape(1, D))


def kernel(x, edge_index, Wqk, bqk, Wv, bv, Wo, bo, gamma, beta):
    src1 = edge_index[0]
    dst1 = edge_index[1]
    pad = jnp.arange(E_PAD - N_EDGES, dtype=jnp.int32) % N_NODES
    dstp = jnp.concatenate([dst1, pad])
    srcp = jnp.concatenate([src1, pad])

    qkb, vlo, vhi = _dense_in(x, Wqk, bqk, Wv, bv)
    qki = jax.lax.bitcast_convert_type(
        qkb.reshape(N_NODES, D // 2, 2), jnp.int32)
    ex = _edge_scores(qki, dstp, srcp)

    zrow = jnp.zeros((RCP, DH), jnp.float32)
    zseg = jnp.zeros((1024,), jnp.float32)
    raw, seg = _aggregate(vlo, vhi, dstp, srcp, ex, zrow, zseg)
    return _dense_out(raw[:N_NODES], raw[N_NODES:], seg[:N_NODES], x, Wo, bo,
                      gamma, beta)


# speedup vs baseline: 1.4180x; 1.4180x over previous
import functools
import math

import jax
import jax.numpy as jnp
from jax import lax
from jax.experimental import pallas as pl
from jax.experimental.pallas import tpu as pltpu
from jax.experimental.pallas import tpu_sc as plsc

N_NODES = 10000
N_EDGES = 160000
D = 256
DH = 128
NC = 2
NS = 16
NW = NC * NS
L = 16

C1 = 64
K1 = 80
NB1 = 3
EW1 = K1 * C1
E_PAD = NW * EW1
C2 = 128
N_CH2 = N_EDGES // C2
RCP = 624

_MESH = plsc.VectorSubcoreMesh(core_axis_name="c", subcore_axis_name="s")

_GDN = lax.GatherDimensionNumbers(
    offset_dims=(), collapsed_slice_dims=(0,), start_index_map=(0,))


def _vgather(x, idx):
    return lax.gather(x, idx[:, None], _GDN, (1,),
                      mode=lax.GatherScatterMode.PROMISE_IN_BOUNDS)


def _dense_in_body(x_ref, wqk_ref, bqk_ref, wv_ref, bv_ref,
                   qkb_ref, vlo_ref, vhi_ref):
    x = x_ref[...]
    qk = jnp.dot(x, wqk_ref[...],
                 preferred_element_type=jnp.float32) + bqk_ref[...]
    qkb_ref[...] = qk.astype(jnp.bfloat16)
    v = jnp.dot(x, wv_ref[...], preferred_element_type=jnp.float32) + bv_ref[...]
    vlo_ref[...] = v[:, :DH]
    vhi_ref[...] = v[:, DH:]


def _dense_in(x, Wqk, bqk, Wv, bv):
    blk = 400
    grid = (N_NODES // blk,)
    return pl.pallas_call(
        _dense_in_body,
        grid=grid,
        in_specs=[
            pl.BlockSpec((blk, D), lambda i: (i, 0)),
            pl.BlockSpec((D, D), lambda i: (0, 0)),
            pl.BlockSpec((1, D), lambda i: (0, 0)),
            pl.BlockSpec((D, D), lambda i: (0, 0)),
            pl.BlockSpec((1, D), lambda i: (0, 0)),
        ],
        out_specs=[
            pl.BlockSpec((blk, D), lambda i: (i, 0)),
            pl.BlockSpec((blk, DH), lambda i: (i, 0)),
            pl.BlockSpec((blk, DH), lambda i: (i, 0)),
        ],
        out_shape=[
            jax.ShapeDtypeStruct((N_NODES, D), jnp.bfloat16),
            jax.ShapeDtypeStruct((N_NODES, DH), jnp.float32),
            jax.ShapeDtypeStruct((N_NODES, DH), jnp.float32),
        ],
    )(x, Wqk, bqk.reshape(1, D), Wv, bv.reshape(1, D))


def _scores_body(qkb_hbm, dst_hbm, src_hbm, ex_hbm,
                 idxd, idxs, qbuf, kbuf, exall, semq, semk):
    wid = lax.axis_index("s") * NC + lax.axis_index("c")
    inv_scale = 1.0 / math.sqrt(D)
    lane = lax.iota(jnp.int32, L)

    pltpu.sync_copy(dst_hbm.at[pl.ds(wid * EW1, EW1)], idxd)
    pltpu.sync_copy(src_hbm.at[pl.ds(wid * EW1, EW1)], idxs)

    def fetch(k, b):
        off = pl.multiple_of(b * C1, 8)
        ioff = k * C1
        pltpu.async_copy(qkb_hbm.at[idxd.at[pl.ds(ioff, C1)]],
                         qbuf.at[pl.ds(off, C1)], semq.at[b])
        pltpu.async_copy(qkb_hbm.at[idxs.at[pl.ds(ioff, C1)]],
                         kbuf.at[pl.ds(off, C1)], semk.at[b])

    fetch(0, 0)
    fetch(1, 1)

    def chunk_step(k, _):
        b = lax.rem(k, NB1)

        @pl.when(k + 2 < K1)
        def _prefetch():
            fetch(k + 2, lax.rem(k + 2, NB1))

        off = pl.multiple_of(b * C1, 8)
        ioff = k * C1
        pltpu.make_async_copy(qkb_hbm.at[idxd.at[pl.ds(ioff, C1)]],
                              qbuf.at[pl.ds(off, C1)], semq.at[b]).wait()
        pltpu.make_async_copy(qkb_hbm.at[idxs.at[pl.ds(ioff, C1)]],
                              kbuf.at[pl.ds(off, C1)], semk.at[b]).wait()

        def grp_step(g, _):
            sv = jnp.zeros((L,), jnp.float32)
            for i in range(L):
                e = off + g * L + i
                terms = []
                for j in range(DH // L):
                    qi = qbuf[e, pl.ds(j * L, L)]
                    ki = kbuf[e, pl.ds(j * L, L)]
                    qa = lax.bitcast_convert_type(qi << 16, jnp.float32)
                    qb = lax.bitcast_convert_type(qi & jnp.int32(-65536),
                                                  jnp.float32)
                    ka = lax.bitcast_convert_type(ki << 16, jnp.float32)
                    kb = lax.bitcast_convert_type(ki & jnp.int32(-65536),
                                                  jnp.float32)
                    terms += [qa * ka, qb * kb]
                while len(terms) > 1:
                    terms = [terms[t] + terms[t + 1]
                             for t in range(0, len(terms), 2)]
                acc = terms[0]
                for sh in (8, 4, 2, 1):
                    acc = acc + _vgather(acc, lane ^ sh)
                sv = jnp.where(lane == i, acc * inv_scale, sv)
            sv = jnp.minimum(jnp.maximum(sv, -5.0), 5.0)
            exall[pl.ds(k * C1 + g * L, L)] = jnp.exp(sv)
            return 0

        lax.fori_loop(0, C1 // L, grp_step, 0)
        return 0

    lax.fori_loop(0, K1, chunk_step, 0)
    pltpu.sync_copy(exall, ex_hbm.at[pl.ds(wid * EW1, EW1)])


def _edge_scores(qkb, dstp, srcp):
    kern = pl.kernel(
        _scores_body,
        out_type=jax.ShapeDtypeStruct((E_PAD,), jnp.float32),
        mesh=_MESH,
        scratch_types=[
            pltpu.VMEM((EW1,), jnp.int32),
            pltpu.VMEM((EW1,), jnp.int32),
            pltpu.VMEM((NB1 * C1, D // 2), jnp.int32),
            pltpu.VMEM((NB1 * C1, D // 2), jnp.int32),
            pltpu.VMEM((EW1,), jnp.float32),
            pltpu.SemaphoreType.DMA((NB1,)),
            pltpu.SemaphoreType.DMA((NB1,)),
        ],
    )
    return kern(qkb, dstp, srcp)


def _aggregate_body(vlo_hbm, vhi_hbm, dst_hbm, src_hbm, ex_hbm, zrow_hbm,
                    zseg_hbm, raw_hbm, seg_hbm,
                    idxd, idxs, vbuf, exbuf, segbuf, raw_sp, seg_sp,
                    semv, semsc, semseg):
    c = lax.axis_index("c")
    sid = lax.axis_index("s")

    pltpu.sync_copy(zrow_hbm, raw_sp.at[pl.ds(sid * RCP, RCP)])

    @pl.when(sid == 0)
    def _zero_tail():
        pltpu.sync_copy(zrow_hbm.at[pl.ds(0, N_NODES - NS * RCP)],
                        raw_sp.at[pl.ds(NS * RCP, N_NODES - NS * RCP)])

    pltpu.sync_copy(zseg_hbm, segbuf.at[0])

    @pl.when(sid < 10)
    def _zero_seg():
        pltpu.sync_copy(segbuf.at[0], seg_sp.at[pl.ds(sid * 1024, 1024)])
    plsc.subcore_barrier()

    n_ch = jnp.where(sid < N_CH2 % NS, N_CH2 // NS + 1, N_CH2 // NS)

    def fetch(k, b):
        r = sid + k * NS
        off = pl.multiple_of(b * C2, 8)
        pltpu.sync_copy(dst_hbm.at[pl.ds(r * C2, C2)], idxd.at[b])
        pltpu.sync_copy(src_hbm.at[pl.ds(r * C2, C2)], idxs.at[b])
        pltpu.sync_copy(ex_hbm.at[pl.ds(r * C2, C2)], exbuf.at[b])

        @pl.when(c == 0)
        def _gather_lo():
            pltpu.async_copy(vlo_hbm.at[idxs.at[b]],
                             vbuf.at[pl.ds(off, C2)], semv.at[b])

        @pl.when(c == 1)
        def _gather_hi():
            pltpu.async_copy(vhi_hbm.at[idxs.at[b]],
                             vbuf.at[pl.ds(off, C2)], semv.at[b])

    fetch(0, 0)

    def chunk_step(k, _):
        b = lax.rem(k, 2)
        nb = 1 - b
        off = pl.multiple_of(b * C2, 8)

        @pl.when(k >= 1)
        def _drain_prev():
            noff = pl.multiple_of(nb * C2, 8)
            pltpu.make_async_copy(vbuf.at[pl.ds(noff, C2)],
                                  raw_sp.at[idxd.at[nb]], semsc.at[nb]).wait()

            @pl.when(c == 0)
            def _drain_seg():
                pltpu.make_async_copy(exbuf.at[nb], seg_sp.at[idxd.at[nb]],
                                      semseg.at[nb]).wait()

        @pl.when(k + 1 < n_ch)
        def _prefetch():
            fetch(k + 1, nb)

        pltpu.make_async_copy(vlo_hbm.at[idxs.at[b]],
                              vbuf.at[pl.ds(off, C2)], semv.at[b]).wait()

        def scale_step(g, _):
            exv = exbuf[b, pl.ds(g * L, L)]
            for i in range(L):
                e = off + g * L + i
                s = _vgather(exv, jnp.full((L,), i, jnp.int32))
                for j in range(DH // L):
                    vbuf[e, pl.ds(j * L, L)] = vbuf[e, pl.ds(j * L, L)] * s
            return 0

        lax.fori_loop(0, C2 // L, scale_step, 0)

        pltpu.async_copy(vbuf.at[pl.ds(off, C2)], raw_sp.at[idxd.at[b]],
                         semsc.at[b], add=True)

        @pl.when(c == 0)
        def _seg_add():
            pltpu.async_copy(exbuf.at[b], seg_sp.at[idxd.at[b]],
                             semseg.at[b], add=True)
        return 0

    lax.fori_loop(0, n_ch, chunk_step, 0)

    bl = lax.rem(n_ch - 1, 2)
    loff = pl.multiple_of(bl * C2, 8)
    pltpu.make_async_copy(vbuf.at[pl.ds(loff, C2)],
                          raw_sp.at[idxd.at[bl]], semsc.at[bl]).wait()

    @pl.when(c == 0)
    def _drain_seg_last():
        pltpu.make_async_copy(exbuf.at[bl], seg_sp.at[idxd.at[bl]],
                              semseg.at[bl]).wait()

    plsc.subcore_barrier()

    pltpu.sync_copy(raw_sp.at[pl.ds(sid * RCP, RCP)],
                    raw_hbm.at[pl.ds(c * N_NODES + sid * RCP, RCP)])

    @pl.when(sid == 0)
    def _raw_tail():
        pltpu.sync_copy(raw_sp.at[pl.ds(NS * RCP, N_NODES - NS * RCP)],
                        raw_hbm.at[pl.ds(c * N_NODES + NS * RCP,
                                         N_NODES - NS * RCP)])

    @pl.when((c == 0) & (sid < 10))
    def _seg_out():
        pltpu.sync_copy(seg_sp.at[pl.ds(sid * 1024, 1024)], segbuf.at[0])
        pltpu.sync_copy(segbuf.at[0], seg_hbm.at[pl.ds(sid * 1024, 1024)])


def _aggregate(vlo, vhi, dst1, src1, ex1, zrow, zseg):
    kern = pl.kernel(
        _aggregate_body,
        out_type=[
            jax.ShapeDtypeStruct((2 * N_NODES, DH), jnp.float32),
            jax.ShapeDtypeStruct((10240,), jnp.float32),
        ],
        mesh=_MESH,
        scratch_types=[
            pltpu.VMEM((2, C2), jnp.int32),
            pltpu.VMEM((2, C2), jnp.int32),
            pltpu.VMEM((2 * C2, DH), jnp.float32),
            pltpu.VMEM((2, C2), jnp.float32),
            pltpu.VMEM((1, 1024), jnp.float32),
            pltpu.VMEM_SHARED((N_NODES, DH), jnp.float32),
            pltpu.VMEM_SHARED((10240,), jnp.float32),
            pltpu.SemaphoreType.DMA((2,)),
            pltpu.SemaphoreType.DMA((2,)),
            pltpu.SemaphoreType.DMA((2,)),
        ],
    )
    return kern(vlo, vhi, dst1, src1, ex1, zrow, zseg)


def _dense_out_body(rawlo_ref, rawhi_ref, seg_ref, x_ref, wo_ref, bo_ref,
                    gamma_ref, beta_ref, out_ref):
    seg = jnp.maximum(seg_ref[...], 1e-30)
    agg = jnp.concatenate([rawlo_ref[...], rawhi_ref[...]], axis=1) / seg
    h = jnp.dot(agg, wo_ref[...], preferred_element_type=jnp.float32)
    h = h + bo_ref[...] + x_ref[...]
    mu = jnp.mean(h, axis=-1, keepdims=True)
    d = h - mu
    var = jnp.mean(d * d, axis=-1, keepdims=True)
    out_ref[...] = d * jax.lax.rsqrt(var + 1e-5) * gamma_ref[...] + beta_ref[...]


def _dense_out(rawlo, rawhi, seg, x, Wo, bo, gamma, beta):
    blk = 400
    grid = (N_NODES // blk,)
    return pl.pallas_call(
        _dense_out_body,
        grid=grid,
        in_specs=[
            pl.BlockSpec((blk, DH), lambda i: (i, 0)),
            pl.BlockSpec((blk, DH), lambda i: (i, 0)),
            pl.BlockSpec((blk, 1), lambda i: (i, 0)),
            pl.BlockSpec((blk, D), lambda i: (i, 0)),
            pl.BlockSpec((D, D), lambda i: (0, 0)),
            pl.BlockSpec((1, D), lambda i: (0, 0)),
            pl.BlockSpec((1, D), lambda i: (0, 0)),
            pl.BlockSpec((1, D), lambda i: (0, 0)),
        ],
        out_specs=pl.BlockSpec((blk, D), lambda i: (i, 0)),
        out_shape=jax.ShapeDtypeStruct((N_NODES, D), jnp.float32),
    )(rawlo, rawhi, seg.reshape(N_NODES, 1), x, Wo, bo.reshape(1, D),
      gamma.reshape(1, D), beta.reshape(1, D))


def kernel(x, edge_index, Wqk, bqk, Wv, bv, Wo, bo, gamma, beta):
    src1 = edge_index[0]
    dst1 = edge_index[1]
    pad = jnp.arange(E_PAD - N_EDGES, dtype=jnp.int32) % N_NODES
    dstp = jnp.concatenate([dst1, pad])
    srcp = jnp.concatenate([src1, pad])

    qkb, vlo, vhi = _dense_in(x, Wqk, bqk, Wv, bv)
    qki = jax.lax.bitcast_convert_type(
        qkb.reshape(N_NODES, D // 2, 2), jnp.int32)
    ex = _edge_scores(qki, dstp, srcp)

    zrow = jnp.zeros((RCP, DH), jnp.float32)
    zseg = jnp.zeros((1024,), jnp.float32)
    raw, seg = _aggregate(vlo, vhi, dstp, srcp, ex, zrow, zseg)
    return _dense_out(raw[:N_NODES], raw[N_NODES:], seg[:N_NODES], x, Wo, bo,
                      gamma, beta)


# speedup vs baseline: 1.4444x; 1.0186x over previous
import functools
import math

import jax
import jax.numpy as jnp
from jax import lax
from jax.experimental import pallas as pl
from jax.experimental.pallas import tpu as pltpu
from jax.experimental.pallas import tpu_sc as plsc

N_NODES = 10000
N_EDGES = 160000
D = 256
DH = 128
NC = 2
NS = 16
NW = NC * NS
L = 16

C1 = 64
K1 = 80
NB1 = 3
EW1 = K1 * C1
E_PAD = NW * EW1
C2 = 128
N_CH2 = N_EDGES // C2
RCP = 624

_MESH = plsc.VectorSubcoreMesh(core_axis_name="c", subcore_axis_name="s")

_GDN = lax.GatherDimensionNumbers(
    offset_dims=(), collapsed_slice_dims=(0,), start_index_map=(0,))


def _vgather(x, idx):
    return lax.gather(x, idx[:, None], _GDN, (1,),
                      mode=lax.GatherScatterMode.PROMISE_IN_BOUNDS)


def _dense_in_body(x_ref, wqk_ref, bqk_ref, wv_ref, bv_ref,
                   qkb_ref, vlo_ref, vhi_ref):
    x = x_ref[...]
    qk = jnp.dot(x, wqk_ref[...],
                 preferred_element_type=jnp.float32) + bqk_ref[...]
    qkb_ref[...] = qk.astype(jnp.bfloat16)
    v = jnp.dot(x, wv_ref[...], preferred_element_type=jnp.float32) + bv_ref[...]
    vlo_ref[...] = v[:, :DH]
    vhi_ref[...] = v[:, DH:]


def _dense_in(x, Wqk, bqk, Wv, bv):
    blk = 400
    grid = (N_NODES // blk,)
    return pl.pallas_call(
        _dense_in_body,
        grid=grid,
        in_specs=[
            pl.BlockSpec((blk, D), lambda i: (i, 0)),
            pl.BlockSpec((D, D), lambda i: (0, 0)),
            pl.BlockSpec((1, D), lambda i: (0, 0)),
            pl.BlockSpec((D, D), lambda i: (0, 0)),
            pl.BlockSpec((1, D), lambda i: (0, 0)),
        ],
        out_specs=[
            pl.BlockSpec((blk, D), lambda i: (i, 0)),
            pl.BlockSpec((blk, DH), lambda i: (i, 0)),
            pl.BlockSpec((blk, DH), lambda i: (i, 0)),
        ],
        out_shape=[
            jax.ShapeDtypeStruct((N_NODES, D), jnp.bfloat16),
            jax.ShapeDtypeStruct((N_NODES, DH), jnp.float32),
            jax.ShapeDtypeStruct((N_NODES, DH), jnp.float32),
        ],
    )(x, Wqk, bqk.reshape(1, D), Wv, bv.reshape(1, D))


def _scores_body(qkb_hbm, dst_hbm, src_hbm, ex_hbm,
                 idxd, idxs, qbuf, kbuf, exall, semq, semk):
    wid = lax.axis_index("s") * NC + lax.axis_index("c")
    inv_scale = 1.0 / math.sqrt(D)
    lane = lax.iota(jnp.int32, L)

    pltpu.sync_copy(dst_hbm.at[pl.ds(wid * EW1, EW1)], idxd)
    pltpu.sync_copy(src_hbm.at[pl.ds(wid * EW1, EW1)], idxs)

    def fetch(k, b):
        off = pl.multiple_of(b * C1, 8)
        ioff = k * C1
        pltpu.async_copy(qkb_hbm.at[idxd.at[pl.ds(ioff, C1)]],
                         qbuf.at[pl.ds(off, C1)], semq.at[b])
        pltpu.async_copy(qkb_hbm.at[idxs.at[pl.ds(ioff, C1)]],
                         kbuf.at[pl.ds(off, C1)], semk.at[b])

    fetch(0, 0)
    fetch(1, 1)

    def chunk_step(k, _):
        b = lax.rem(k, NB1)

        @pl.when(k + 2 < K1)
        def _prefetch():
            fetch(k + 2, lax.rem(k + 2, NB1))

        off = pl.multiple_of(b * C1, 8)
        ioff = k * C1
        pltpu.make_async_copy(qkb_hbm.at[idxd.at[pl.ds(ioff, C1)]],
                              qbuf.at[pl.ds(off, C1)], semq.at[b]).wait()
        pltpu.make_async_copy(qkb_hbm.at[idxs.at[pl.ds(ioff, C1)]],
                              kbuf.at[pl.ds(off, C1)], semk.at[b]).wait()

        def grp_step(g, _):
            sv = jnp.zeros((L,), jnp.float32)
            for i in range(L):
                e = off + g * L + i
                terms = []
                for j in range(DH // L):
                    qi = qbuf[e, pl.ds(j * L, L)]
                    ki = kbuf[e, pl.ds(j * L, L)]
                    qa = lax.bitcast_convert_type(qi << 16, jnp.float32)
                    qb = lax.bitcast_convert_type(qi & jnp.int32(-65536),
                                                  jnp.float32)
                    ka = lax.bitcast_convert_type(ki << 16, jnp.float32)
                    kb = lax.bitcast_convert_type(ki & jnp.int32(-65536),
                                                  jnp.float32)
                    terms += [qa * ka, qb * kb]
                while len(terms) > 1:
                    terms = [terms[t] + terms[t + 1]
                             for t in range(0, len(terms), 2)]
                acc = terms[0]
                for sh in (8, 4, 2, 1):
                    acc = acc + _vgather(acc, lane ^ sh)
                sv = jnp.where(lane == i, acc * inv_scale, sv)
            sv = jnp.minimum(jnp.maximum(sv, -5.0), 5.0)
            exall[pl.ds(k * C1 + g * L, L)] = jnp.exp(sv)
            return 0

        lax.fori_loop(0, C1 // L, grp_step, 0)
        return 0

    lax.fori_loop(0, K1, chunk_step, 0)
    pltpu.sync_copy(exall, ex_hbm.at[pl.ds(wid * EW1, EW1)])


def _edge_scores(qkb, dstp, srcp):
    kern = pl.kernel(
        _scores_body,
        out_type=jax.ShapeDtypeStruct((E_PAD,), jnp.float32),
        mesh=_MESH,
        scratch_types=[
            pltpu.VMEM((EW1,), jnp.int32),
            pltpu.VMEM((EW1,), jnp.int32),
            pltpu.VMEM((NB1 * C1, D // 2), jnp.int32),
            pltpu.VMEM((NB1 * C1, D // 2), jnp.int32),
            pltpu.VMEM((EW1,), jnp.float32),
            pltpu.SemaphoreType.DMA((NB1,)),
            pltpu.SemaphoreType.DMA((NB1,)),
        ],
    )
    return kern(qkb, dstp, srcp)


def _aggregate_body(vlo_hbm, vhi_hbm, dst_hbm, src_hbm, ex_hbm, zrow_hbm,
                    zseg_hbm, raw_hbm, seg_hbm,
                    idxd, idxs, vbuf, exbuf, segbuf, raw_sp, seg_sp,
                    semv, semsc, semseg):
    c = lax.axis_index("c")
    sid = lax.axis_index("s")

    pltpu.sync_copy(zrow_hbm, raw_sp.at[pl.ds(sid * RCP, RCP)])

    @pl.when(sid == 0)
    def _zero_tail():
        pltpu.sync_copy(zrow_hbm.at[pl.ds(0, N_NODES - NS * RCP)],
                        raw_sp.at[pl.ds(NS * RCP, N_NODES - NS * RCP)])

    pltpu.sync_copy(zseg_hbm, segbuf.at[0])

    @pl.when(sid < 10)
    def _zero_seg():
        pltpu.sync_copy(segbuf.at[0], seg_sp.at[pl.ds(sid * 1024, 1024)])
    plsc.subcore_barrier()

    n_ch = jnp.where(sid < N_CH2 % NS, N_CH2 // NS + 1, N_CH2 // NS)

    def fetch(k, b):
        r = sid + k * NS
        off = pl.multiple_of(b * C2, 8)
        pltpu.sync_copy(dst_hbm.at[pl.ds(r * C2, C2)], idxd.at[b])
        pltpu.sync_copy(src_hbm.at[pl.ds(r * C2, C2)], idxs.at[b])
        pltpu.sync_copy(ex_hbm.at[pl.ds(r * C2, C2)], exbuf.at[b])

        @pl.when(c == 0)
        def _gather_lo():
            pltpu.async_copy(vlo_hbm.at[idxs.at[b]],
                             vbuf.at[pl.ds(off, C2)], semv.at[b])

        @pl.when(c == 1)
        def _gather_hi():
            pltpu.async_copy(vhi_hbm.at[idxs.at[b]],
                             vbuf.at[pl.ds(off, C2)], semv.at[b])

    fetch(0, 0)

    def chunk_step(k, _):
        b = lax.rem(k, 2)
        nb = 1 - b
        off = pl.multiple_of(b * C2, 8)

        @pl.when(k >= 1)
        def _drain_prev():
            noff = pl.multiple_of(nb * C2, 8)
            pltpu.make_async_copy(vbuf.at[pl.ds(noff, C2)],
                                  raw_sp.at[idxd.at[nb]], semsc.at[nb]).wait()

            @pl.when(c == 0)
            def _drain_seg():
                pltpu.make_async_copy(exbuf.at[nb], seg_sp.at[idxd.at[nb]],
                                      semseg.at[nb]).wait()

        @pl.when(k + 1 < n_ch)
        def _prefetch():
            fetch(k + 1, nb)

        pltpu.make_async_copy(vlo_hbm.at[idxs.at[b]],
                              vbuf.at[pl.ds(off, C2)], semv.at[b]).wait()

        def scale_step(g, _):
            exv = exbuf[b, pl.ds(g * L, L)]
            for i in range(L):
                e = off + g * L + i
                s = _vgather(exv, jnp.full((L,), i, jnp.int32))
                for j in range(DH // L):
                    vbuf[e, pl.ds(j * L, L)] = vbuf[e, pl.ds(j * L, L)] * s
            return 0

        lax.fori_loop(0, C2 // L, scale_step, 0)

        pltpu.async_copy(vbuf.at[pl.ds(off, C2)], raw_sp.at[idxd.at[b]],
                         semsc.at[b], add=True)

        @pl.when(c == 0)
        def _seg_add():
            pltpu.async_copy(exbuf.at[b], seg_sp.at[idxd.at[b]],
                             semseg.at[b], add=True)
        return 0

    lax.fori_loop(0, n_ch, chunk_step, 0)

    bl = lax.rem(n_ch - 1, 2)
    loff = pl.multiple_of(bl * C2, 8)
    pltpu.make_async_copy(vbuf.at[pl.ds(loff, C2)],
                          raw_sp.at[idxd.at[bl]], semsc.at[bl]).wait()

    @pl.when(c == 0)
    def _drain_seg_last():
        pltpu.make_async_copy(exbuf.at[bl], seg_sp.at[idxd.at[bl]],
                              semseg.at[bl]).wait()

    plsc.subcore_barrier()

    pltpu.sync_copy(raw_sp.at[pl.ds(sid * RCP, RCP)],
                    raw_hbm.at[pl.ds(c * N_NODES + sid * RCP, RCP)])

    @pl.when(sid == 0)
    def _raw_tail():
        pltpu.sync_copy(raw_sp.at[pl.ds(NS * RCP, N_NODES - NS * RCP)],
                        raw_hbm.at[pl.ds(c * N_NODES + NS * RCP,
                                         N_NODES - NS * RCP)])

    @pl.when((c == 0) & (sid < 10))
    def _seg_out():
        pltpu.sync_copy(seg_sp.at[pl.ds(sid * 1024, 1024)], segbuf.at[0])
        pltpu.sync_copy(segbuf.at[0], seg_hbm.at[pl.ds(sid * 1024, 1024)])


def _aggregate(vlo, vhi, dst1, src1, ex1, zrow, zseg):
    kern = pl.kernel(
        _aggregate_body,
        out_type=[
            jax.ShapeDtypeStruct((2 * N_NODES, DH), jnp.float32),
            jax.ShapeDtypeStruct((10240,), jnp.float32),
        ],
        mesh=_MESH,
        scratch_types=[
            pltpu.VMEM((2, C2), jnp.int32),
            pltpu.VMEM((2, C2), jnp.int32),
            pltpu.VMEM((2 * C2, DH), jnp.float32),
            pltpu.VMEM((2, C2), jnp.float32),
            pltpu.VMEM((1, 1024), jnp.float32),
            pltpu.VMEM_SHARED((N_NODES, DH), jnp.float32),
            pltpu.VMEM_SHARED((10240,), jnp.float32),
            pltpu.SemaphoreType.DMA((2,)),
            pltpu.SemaphoreType.DMA((2,)),
            pltpu.SemaphoreType.DMA((2,)),
        ],
    )
    return kern(vlo, vhi, dst1, src1, ex1, zrow, zseg)


def _dense_out_body(rawlo_ref, rawhi_ref, seg_ref, x_ref, wo_ref, bo_ref,
                    gamma_ref, beta_ref, out_ref):
    seg = jnp.maximum(seg_ref[...], 1e-30)
    agg = jnp.concatenate([rawlo_ref[...], rawhi_ref[...]], axis=1) / seg
    h = jnp.dot(agg, wo_ref[...], preferred_element_type=jnp.float32)
    h = h + bo_ref[...] + x_ref[...]
    mu = jnp.mean(h, axis=-1, keepdims=True)
    d = h - mu
    var = jnp.mean(d * d, axis=-1, keepdims=True)
    out_ref[...] = d * jax.lax.rsqrt(var + 1e-5) * gamma_ref[...] + beta_ref[...]


def _dense_out(raw, seg, x, Wo, bo, gamma, beta):
    blk = 400
    nb = N_NODES // blk
    grid = (nb,)
    return pl.pallas_call(
        _dense_out_body,
        grid=grid,
        in_specs=[
            pl.BlockSpec((blk, DH), lambda i: (i, 0)),
            pl.BlockSpec((blk, DH), lambda i: (i + N_NODES // 400, 0)),
            pl.BlockSpec((blk, 1), lambda i: (i, 0)),
            pl.BlockSpec((blk, D), lambda i: (i, 0)),
            pl.BlockSpec((D, D), lambda i: (0, 0)),
            pl.BlockSpec((1, D), lambda i: (0, 0)),
            pl.BlockSpec((1, D), lambda i: (0, 0)),
            pl.BlockSpec((1, D), lambda i: (0, 0)),
        ],
        out_specs=pl.BlockSpec((blk, D), lambda i: (i, 0)),
        out_shape=jax.ShapeDtypeStruct((N_NODES, D), jnp.float32),
    )(raw, raw, seg.reshape(10240, 1), x, Wo, bo.reshape(1, D),
      gamma.reshape(1, D), beta.reshape(1, D))


def kernel(x, edge_index, Wqk, bqk, Wv, bv, Wo, bo, gamma, beta):
    src1 = edge_index[0]
    dst1 = edge_index[1]
    pad = jnp.arange(E_PAD - N_EDGES, dtype=jnp.int32) % N_NODES
    dstp = jnp.concatenate([dst1, pad])
    srcp = jnp.concatenate([src1, pad])

    qkb, vlo, vhi = _dense_in(x, Wqk, bqk, Wv, bv)
    qki = jax.lax.bitcast_convert_type(
        qkb.reshape(N_NODES, D // 2, 2), jnp.int32)
    ex = _edge_scores(qki, dstp, srcp)

    zrow = jnp.zeros((RCP, DH), jnp.float32)
    zseg = jnp.zeros((1024,), jnp.float32)
    raw, seg = _aggregate(vlo, vhi, dstp, srcp, ex, zrow, zseg)
    return _dense_out(raw, seg, x, Wo, bo, gamma, beta)


# speedup vs baseline: 1.4449x; 1.0003x over previous
import math

import jax
import jax.numpy as jnp
from jax import lax
from jax.experimental import pallas as pl
from jax.experimental.pallas import tpu as pltpu
from jax.experimental.pallas import tpu_sc as plsc

N_NODES = 10000
N_EDGES = 160000
D = 256
DH = 128
NC = 2
NS = 16
NW = NC * NS
L = 16

C1 = 64
K1 = 80
NB1 = 3
EW1 = K1 * C1
E_PAD = NW * EW1
C2 = 128
N_CH2 = N_EDGES // C2
RCP = 624

_MESH = plsc.VectorSubcoreMesh(core_axis_name="c", subcore_axis_name="s")

_GDN = lax.GatherDimensionNumbers(
    offset_dims=(), collapsed_slice_dims=(0,), start_index_map=(0,))


def _vgather(x, idx):
    return lax.gather(x, idx[:, None], _GDN, (1,),
                      mode=lax.GatherScatterMode.PROMISE_IN_BOUNDS)


def _dense_in_body(x_ref, wqk_ref, bqk_ref, wv_ref, bv_ref,
                   qkb_ref, vlo_ref, vhi_ref):
    x = x_ref[...]
    qk = jnp.dot(x, wqk_ref[...],
                 preferred_element_type=jnp.float32) + bqk_ref[...]
    qkb_ref[...] = qk.astype(jnp.bfloat16)
    v = jnp.dot(x, wv_ref[...], preferred_element_type=jnp.float32) + bv_ref[...]
    vlo_ref[...] = v[:, :DH]
    vhi_ref[...] = v[:, DH:]


def _dense_in(x, Wqk, bqk, Wv, bv):
    blk = 400
    grid = (N_NODES // blk,)
    return pl.pallas_call(
        _dense_in_body,
        grid=grid,
        in_specs=[
            pl.BlockSpec((blk, D), lambda i: (i, 0)),
            pl.BlockSpec((D, D), lambda i: (0, 0)),
            pl.BlockSpec((1, D), lambda i: (0, 0)),
            pl.BlockSpec((D, D), lambda i: (0, 0)),
            pl.BlockSpec((1, D), lambda i: (0, 0)),
        ],
        out_specs=[
            pl.BlockSpec((blk, D), lambda i: (i, 0)),
            pl.BlockSpec((blk, DH), lambda i: (i, 0)),
            pl.BlockSpec((blk, DH), lambda i: (i, 0)),
        ],
        out_shape=[
            jax.ShapeDtypeStruct((N_NODES, D), jnp.bfloat16),
            jax.ShapeDtypeStruct((N_NODES, DH), jnp.float32),
            jax.ShapeDtypeStruct((N_NODES, DH), jnp.float32),
        ],
    )(x, Wqk, bqk.reshape(1, D), Wv, bv.reshape(1, D))


def _scores_body(qkb_hbm, dst_hbm, src_hbm, ex_hbm,
                 idxd, idxs, qbuf, kbuf, exall, semq, semk):
    wid = lax.axis_index("s") * NC + lax.axis_index("c")
    inv_scale = 1.0 / math.sqrt(D)
    lane = lax.iota(jnp.int32, L)

    pltpu.sync_copy(dst_hbm.at[pl.ds(wid * EW1, EW1)], idxd)
    pltpu.sync_copy(src_hbm.at[pl.ds(wid * EW1, EW1)], idxs)

    def fetch(k, b):
        off = pl.multiple_of(b * C1, 8)
        ioff = k * C1
        pltpu.async_copy(qkb_hbm.at[idxd.at[pl.ds(ioff, C1)]],
                         qbuf.at[pl.ds(off, C1)], semq.at[b])
        pltpu.async_copy(qkb_hbm.at[idxs.at[pl.ds(ioff, C1)]],
                         kbuf.at[pl.ds(off, C1)], semk.at[b])

    fetch(0, 0)
    fetch(1, 1)

    def chunk_step(k, _):
        b = lax.rem(k, NB1)

        @pl.when(k + 2 < K1)
        def _prefetch():
            fetch(k + 2, lax.rem(k + 2, NB1))

        off = pl.multiple_of(b * C1, 8)
        ioff = k * C1
        pltpu.make_async_copy(qkb_hbm.at[idxd.at[pl.ds(ioff, C1)]],
                              qbuf.at[pl.ds(off, C1)], semq.at[b]).wait()
        pltpu.make_async_copy(qkb_hbm.at[idxs.at[pl.ds(ioff, C1)]],
                              kbuf.at[pl.ds(off, C1)], semk.at[b]).wait()

        def grp_step(g, _):
            sv = jnp.zeros((L,), jnp.float32)
            for i in range(L):
                e = off + g * L + i
                terms = []
                for j in range(DH // L):
                    qi = qbuf[e, pl.ds(j * L, L)]
                    ki = kbuf[e, pl.ds(j * L, L)]
                    qa = lax.bitcast_convert_type(qi << 16, jnp.float32)
                    qb = lax.bitcast_convert_type(qi & jnp.int32(-65536),
                                                  jnp.float32)
                    ka = lax.bitcast_convert_type(ki << 16, jnp.float32)
                    kb = lax.bitcast_convert_type(ki & jnp.int32(-65536),
                                                  jnp.float32)
                    terms += [qa * ka, qb * kb]
                while len(terms) > 1:
                    terms = [terms[t] + terms[t + 1]
                             for t in range(0, len(terms), 2)]
                acc = terms[0]
                for sh in (8, 4, 2, 1):
                    acc = acc + _vgather(acc, lane ^ sh)
                sv = jnp.where(lane == i, acc * inv_scale, sv)
            sv = jnp.minimum(jnp.maximum(sv, -5.0), 5.0)
            exall[pl.ds(k * C1 + g * L, L)] = jnp.exp(sv)
            return 0

        lax.fori_loop(0, C1 // L, grp_step, 0)
        return 0

    lax.fori_loop(0, K1, chunk_step, 0)
    pltpu.sync_copy(exall, ex_hbm.at[pl.ds(wid * EW1, EW1)])


def _edge_scores(qkb, dstp, srcp):
    kern = pl.kernel(
        _scores_body,
        out_type=jax.ShapeDtypeStruct((E_PAD,), jnp.float32),
        mesh=_MESH,
        scratch_types=[
            pltpu.VMEM((EW1,), jnp.int32),
            pltpu.VMEM((EW1,), jnp.int32),
            pltpu.VMEM((NB1 * C1, D // 2), jnp.int32),
            pltpu.VMEM((NB1 * C1, D // 2), jnp.int32),
            pltpu.VMEM((EW1,), jnp.float32),
            pltpu.SemaphoreType.DMA((NB1,)),
            pltpu.SemaphoreType.DMA((NB1,)),
        ],
    )
    return kern(qkb, dstp, srcp)


def _aggregate_body(vlo_hbm, vhi_hbm, dst_hbm, src_hbm, ex_hbm, zrow_hbm,
                    zseg_hbm, raw_hbm, seg_hbm,
                    idxd, idxs, vbuf, exbuf, segbuf, raw_sp, seg_sp,
                    semv, semsc, semseg):
    c = lax.axis_index("c")
    sid = lax.axis_index("s")

    pltpu.sync_copy(zrow_hbm, raw_sp.at[pl.ds(sid * RCP, RCP)])

    @pl.when(sid == 0)
    def _zero_tail():
        pltpu.sync_copy(zrow_hbm.at[pl.ds(0, N_NODES - NS * RCP)],
                        raw_sp.at[pl.ds(NS * RCP, N_NODES - NS * RCP)])

    pltpu.sync_copy(zseg_hbm, segbuf.at[0])

    @pl.when(sid < 10)
    def _zero_seg():
        pltpu.sync_copy(segbuf.at[0], seg_sp.at[pl.ds(sid * 1024, 1024)])
    plsc.subcore_barrier()

    n_ch = jnp.where(sid < N_CH2 % NS, N_CH2 // NS + 1, N_CH2 // NS)

    def fetch(k, b):
        r = sid + k * NS
        off = pl.multiple_of(b * C2, 8)
        pltpu.sync_copy(dst_hbm.at[pl.ds(r * C2, C2)], idxd.at[b])
        pltpu.sync_copy(src_hbm.at[pl.ds(r * C2, C2)], idxs.at[b])
        pltpu.sync_copy(ex_hbm.at[pl.ds(r * C2, C2)], exbuf.at[b])

        @pl.when(c == 0)
        def _gather_lo():
            pltpu.async_copy(vlo_hbm.at[idxs.at[b]],
                             vbuf.at[pl.ds(off, C2)], semv.at[b])

        @pl.when(c == 1)
        def _gather_hi():
            pltpu.async_copy(vhi_hbm.at[idxs.at[b]],
                             vbuf.at[pl.ds(off, C2)], semv.at[b])

    fetch(0, 0)

    def chunk_step(k, _):
        b = lax.rem(k, 2)
        nb = 1 - b
        off = pl.multiple_of(b * C2, 8)

        @pl.when(k >= 1)
        def _drain_prev():
            noff = pl.multiple_of(nb * C2, 8)
            pltpu.make_async_copy(vbuf.at[pl.ds(noff, C2)],
                                  raw_sp.at[idxd.at[nb]], semsc.at[nb]).wait()

            @pl.when(c == 0)
            def _drain_seg():
                pltpu.make_async_copy(exbuf.at[nb], seg_sp.at[idxd.at[nb]],
                                      semseg.at[nb]).wait()

        @pl.when(k + 1 < n_ch)
        def _prefetch():
            fetch(k + 1, nb)

        pltpu.make_async_copy(vlo_hbm.at[idxs.at[b]],
                              vbuf.at[pl.ds(off, C2)], semv.at[b]).wait()

        def scale_step(g, _):
            exv = exbuf[b, pl.ds(g * L, L)]
            for i in range(L):
                e = off + g * L + i
                s = _vgather(exv, jnp.full((L,), i, jnp.int32))
                for j in range(DH // L):
                    vbuf[e, pl.ds(j * L, L)] = vbuf[e, pl.ds(j * L, L)] * s
            return 0

        lax.fori_loop(0, C2 // L, scale_step, 0)

        pltpu.async_copy(vbuf.at[pl.ds(off, C2)], raw_sp.at[idxd.at[b]],
                         semsc.at[b], add=True)

        @pl.when(c == 0)
        def _seg_add():
            pltpu.async_copy(exbuf.at[b], seg_sp.at[idxd.at[b]],
                             semseg.at[b], add=True)
        return 0

    lax.fori_loop(0, n_ch, chunk_step, 0)

    bl = lax.rem(n_ch - 1, 2)
    loff = pl.multiple_of(bl * C2, 8)
    pltpu.make_async_copy(vbuf.at[pl.ds(loff, C2)],
                          raw_sp.at[idxd.at[bl]], semsc.at[bl]).wait()

    @pl.when(c == 0)
    def _drain_seg_last():
        pltpu.make_async_copy(exbuf.at[bl], seg_sp.at[idxd.at[bl]],
                              semseg.at[bl]).wait()

    plsc.subcore_barrier()

    pltpu.sync_copy(raw_sp.at[pl.ds(sid * RCP, RCP)],
                    raw_hbm.at[pl.ds(c * N_NODES + sid * RCP, RCP)])

    @pl.when(sid == 0)
    def _raw_tail():
        pltpu.sync_copy(raw_sp.at[pl.ds(NS * RCP, N_NODES - NS * RCP)],
                        raw_hbm.at[pl.ds(c * N_NODES + NS * RCP,
                                         N_NODES - NS * RCP)])

    @pl.when((c == 0) & (sid < 10))
    def _seg_out():
        pltpu.sync_copy(seg_sp.at[pl.ds(sid * 1024, 1024)], segbuf.at[0])
        pltpu.sync_copy(segbuf.at[0], seg_hbm.at[pl.ds(sid * 1024, 1024)])


def _aggregate(vlo, vhi, dst1, src1, ex1, zrow, zseg):
    kern = pl.kernel(
        _aggregate_body,
        out_type=[
            jax.ShapeDtypeStruct((2 * N_NODES, DH), jnp.float32),
            jax.ShapeDtypeStruct((10240,), jnp.float32),
        ],
        mesh=_MESH,
        scratch_types=[
            pltpu.VMEM((2, C2), jnp.int32),
            pltpu.VMEM((2, C2), jnp.int32),
            pltpu.VMEM((2 * C2, DH), jnp.float32),
            pltpu.VMEM((2, C2), jnp.float32),
            pltpu.VMEM((1, 1024), jnp.float32),
            pltpu.VMEM_SHARED((N_NODES, DH), jnp.float32),
            pltpu.VMEM_SHARED((10240,), jnp.float32),
            pltpu.SemaphoreType.DMA((2,)),
            pltpu.SemaphoreType.DMA((2,)),
            pltpu.SemaphoreType.DMA((2,)),
        ],
    )
    return kern(vlo, vhi, dst1, src1, ex1, zrow, zseg)


def _dense_out_body(rawlo_ref, rawhi_ref, seg_ref, x_ref, wo_ref, bo_ref,
                    gamma_ref, beta_ref, out_ref):
    seg = jnp.maximum(seg_ref[...], 1e-30)
    agg = jnp.concatenate([rawlo_ref[...], rawhi_ref[...]], axis=1) / seg
    h = jnp.dot(agg, wo_ref[...], preferred_element_type=jnp.float32)
    h = h + bo_ref[...] + x_ref[...]
    mu = jnp.mean(h, axis=-1, keepdims=True)
    d = h - mu
    var = jnp.mean(d * d, axis=-1, keepdims=True)
    out_ref[...] = d * jax.lax.rsqrt(var + 1e-5) * gamma_ref[...] + beta_ref[...]


def _dense_out(raw, seg, x, Wo, bo, gamma, beta):
    blk = 400
    nb = N_NODES // blk
    grid = (nb,)
    return pl.pallas_call(
        _dense_out_body,
        grid=grid,
        in_specs=[
            pl.BlockSpec((blk, DH), lambda i: (i, 0)),
            pl.BlockSpec((blk, DH), lambda i: (i + N_NODES // 400, 0)),
            pl.BlockSpec((blk, 1), lambda i: (i, 0)),
            pl.BlockSpec((blk, D), lambda i: (i, 0)),
            pl.BlockSpec((D, D), lambda i: (0, 0)),
            pl.BlockSpec((1, D), lambda i: (0, 0)),
            pl.BlockSpec((1, D), lambda i: (0, 0)),
            pl.BlockSpec((1, D), lambda i: (0, 0)),
        ],
        out_specs=pl.BlockSpec((blk, D), lambda i: (i, 0)),
        out_shape=jax.ShapeDtypeStruct((N_NODES, D), jnp.float32),
    )(raw, raw, seg.reshape(10240, 1), x, Wo, bo.reshape(1, D),
      gamma.reshape(1, D), beta.reshape(1, D))


def kernel(x, edge_index, Wqk, bqk, Wv, bv, Wo, bo, gamma, beta):
    src1 = edge_index[0]
    dst1 = edge_index[1]
    pad = jnp.arange(E_PAD - N_EDGES, dtype=jnp.int32) % N_NODES
    dstp = jnp.concatenate([dst1, pad])
    srcp = jnp.concatenate([src1, pad])

    qkb, vlo, vhi = _dense_in(x, Wqk, bqk, Wv, bv)
    qki = jax.lax.bitcast_convert_type(
        qkb.reshape(N_NODES, D // 2, 2), jnp.int32)
    ex = _edge_scores(qki, dstp, srcp)

    zrow = jnp.zeros((RCP, DH), jnp.float32)
    zseg = jnp.zeros((1024,), jnp.float32)
    raw, seg = _aggregate(vlo, vhi, dstp, srcp, ex, zrow, zseg)
    return _dense_out(raw, seg, x, Wo, bo, gamma, beta)
